# Initial kernel scaffold; baseline (speedup 1.0000x reference)
import jax
import jax.numpy as jnp
from jax.experimental import pallas as pl


def kernel(node_features, edge_features, adj):
    raise NotImplementedError("write your pallas kernel here")


# speedup vs baseline: 1.4742x; 1.4742x over previous
import functools

import jax
import jax.numpy as jnp
from jax.experimental import pallas as pl

N = 4096
D = 512
BI = 512
BK = 512


def _body(nf_ref, e_ref, a_ref, o_ref):
    i = pl.program_id(0)
    k = pl.program_id(1)
    a = a_ref[...]

    @pl.when(k == 0)
    def _init():
        o_ref[:, :D] = nf_ref[pl.ds(i * BI, BI), :]
        o_ref[:, D:] = jnp.zeros((BI, D + 1), jnp.float32)

    nf_k = nf_ref[pl.ds(k * BK, BK), :]
    o_ref[:, D:2 * D] += jax.lax.dot(a, nf_k, preferred_element_type=jnp.float32)
    o_ref[:, 2 * D:] += jnp.sum(a * e_ref[...].T, axis=1, keepdims=True)


@jax.jit
def kernel(node_features, edge_features, adj):
    grid = (N // BI, N // BK)
    return pl.pallas_call(
        _body,
        grid=grid,
        in_specs=[
            pl.BlockSpec((N, D), lambda i, k: (0, 0)),
            pl.BlockSpec((BK, BI), lambda i, k: (k, i)),
            pl.BlockSpec((BI, BK), lambda i, k: (i, k)),
        ],
        out_specs=pl.BlockSpec((BI, 2 * D + 1), lambda i, k: (i, 0)),
        out_shape=jax.ShapeDtypeStruct((N, 2 * D + 1), jnp.float32),
    )(node_features, edge_features, adj)
